# baseline (speedup 1.0000x reference)
import functools

import jax
import jax.numpy as jnp
from jax import lax
from jax.experimental import pallas as pl
from jax.experimental.pallas import tpu as pltpu
from jax.experimental.pallas import tpu_sc as plsc

WINDOW = 61
PSIZE = 4
IMG = 64
NQ = 961
NQ_PAD = 976
T = PSIZE * PSIZE
OUT_ROWS = NQ * T
LANES = 16
NW = 32
CQ = 4
CROWS = CQ * T
NCHUNK = 8
LB = 4
CGROWS = CROWS * LB
EB0 = 28 * IMG


def _make_tr_body(nchunks, chunk_rows):
    ce = chunk_rows * IMG

    def _tr_body(x4_hbm, xt_hbm, xv0, xv1, tb0, tb1,
                 in_sem0, in_sem1, out_sem0, out_sem1):
        nc = 2
        wid = lax.axis_index("s") * nc + lax.axis_index("c")
        b = wid // 2
        c0 = (wid % 2) * 16
        tn = wid // 8
        ln0 = (wid % 8) * 16

        iota = lax.broadcasted_iota(jnp.int32, (LANES,), 0)
        kv = [jnp.broadcast_to(jnp.int32(k), (LANES,)) for k in range(16)]

        xvs = (xv0, xv1)
        tbs = (tb0, tb1)
        in_sems = (in_sem0, in_sem1)
        out_sems = (out_sem0, out_sem1)

        def start_in(ch, bb):
            cp = pltpu.make_async_copy(
                x4_hbm.at[b, pl.ds(c0, 16), pl.ds(ch * chunk_rows,
                                                  chunk_rows)],
                xvs[bb], in_sems[bb])
            cp.start()
            return cp

        copies_in = [None, None]
        copies_out = [None, None]
        copies_in[0] = start_in(0, 0)
        for ch in range(nchunks):
            bb = ch % 2
            nb = (ch + 1) % 2
            if ch + 1 < nchunks:
                if copies_out[nb] is not None:
                    copies_out[nb].wait()
                    copies_out[nb] = None
                copies_in[nb] = start_in(ch + 1, nb)
            copies_in[bb].wait()
            xv = xvs[bb]
            tb = tbs[bb]

            @plsc.parallel_loop(0, ce // LANES, unroll=4)
            def eg_step(eg):
                e0 = eg * LANES
                rr = lax.shift_right_logical(e0, 6)
                cc0 = e0 & 63
                ev = e0 + iota
                for k in range(16):
                    vals = xv[k, rr, pl.ds(cc0, LANES)]
                    plsc.store_scatter(tb, [ev, kv[k]], vals)

            copies_out[bb] = pltpu.make_async_copy(
                tb, xt_hbm.at[pl.ds(ch * ce, ce), tn, pl.ds(ln0, LANES)],
                out_sems[bb])
            copies_out[bb].start()
        for bb in range(2):
            if copies_out[bb] is not None:
                copies_out[bb].wait()

    return _tr_body


def _sc_body(xta_hbm, xtb_hbm, mask_hbm, out_hbm, mask_v, src_v, src2_v,
             buf0, buf1, buf2,
             in_sem0, in_sem1, in_sem2, out_sem0, out_sem1, out_sem2):
    nc = 2
    wid = lax.axis_index("s") * nc + lax.axis_index("c")
    q_start = (NQ * wid) // NW
    q_end = (NQ * (wid + 1)) // NW

    iota = lax.broadcasted_iota(jnp.int32, (LANES,), 0)
    off_v = lax.shift_right_logical(iota, 2) * IMG + (iota & 3)

    pltpu.sync_copy(mask_hbm, mask_v)

    def chunk_q0(c):
        return jnp.minimum(q_start + CQ * c, q_end - CQ)

    def chunk_in_b(c):
        qc = chunk_q0(c)
        return lax.shift_right_logical(qc * 67651, 21) >= 14

    for c in range(NCHUNK):
        qc = chunk_q0(c)
        bias = jnp.where(chunk_in_b(c), jnp.int32(EB0), jnp.int32(0))
        for gq in range(CQ):
            qv = jnp.broadcast_to(qc + gq, (LANES,))
            w = plsc.load_gather(mask_v, [qv])
            rr = lax.shift_right_logical(w * 68760, 22)
            base = w + 3 * rr
            src_v[c, pl.ds(gq * LANES, LANES)] = base + off_v - bias

    for c in range(NCHUNK):
        def build(jg, _, c=c):
            jv = jg * LANES + iota
            tp = lax.shift_right_logical(jv, 5)
            tn = lax.shift_right_logical(jv, 3) & 3
            sp = jv & 7
            p_local = tp * 8 + sp
            cv = jnp.broadcast_to(jnp.int32(c), (LANES,))
            sr = plsc.load_gather(src_v, [cv, p_local])
            h = lax.shift_right_logical(jv, 7)
            jr = jv & 127
            plsc.store_scatter(src2_v, [2 * c + h, jr], sr * LB + tn)
            return 0

        lax.fori_loop(0, CGROWS // LANES, build, 0)

    bufs = (buf0, buf1, buf2)
    in_sems = (in_sem0, in_sem1, in_sem2)
    out_sems = (out_sem0, out_sem1, out_sem2)

    def band0(c):
        qc = jnp.minimum(q_start + CQ * c, q_end - CQ)
        return qc * (T // 8) * (LB * 8)

    def gather_copies(c, b, table):
        return (
            pltpu.make_async_copy(
                table.at[src2_v.at[2 * c]],
                bufs[b].at[pl.ds(0, 128)], in_sems[b]),
            pltpu.make_async_copy(
                table.at[src2_v.at[2 * c + 1]],
                bufs[b].at[pl.ds(128, 128)], in_sems[b]),
        )

    def start_gather(c, b):
        use_b = chunk_in_b(c)

        @pl.when(use_b)
        def _():
            for cp in gather_copies(c, b, xtb_hbm):
                cp.start()

        @pl.when(jnp.logical_not(use_b))
        def _():
            for cp in gather_copies(c, b, xta_hbm):
                cp.start()

    def wait_gather(c, b):
        for cp in gather_copies(c, b, xta_hbm):
            cp.wait()

    copies_out = [None, None, None]
    start_gather(0, 0)
    start_gather(1, 1)
    for c in range(NCHUNK):
        b = c % 3
        if c + 2 < NCHUNK:
            pb = (c + 2) % 3
            if copies_out[pb] is not None:
                copies_out[pb].wait()
                copies_out[pb] = None
            start_gather(c + 2, pb)
        wait_gather(c, b)
        copies_out[b] = pltpu.make_async_copy(
            bufs[b], out_hbm.at[pl.ds(band0(c), CGROWS)], out_sems[b])
        copies_out[b].start()
    for b in range(3):
        if copies_out[b] is not None:
            copies_out[b].wait()


def kernel(input_data, mask):
    B, C, H, W = input_data.shape
    rows = B * C
    mask_p = jnp.pad(mask.astype(jnp.int32), (0, NQ_PAD - NQ))

    mesh = plsc.VectorSubcoreMesh(core_axis_name="c", subcore_axis_name="s")

    def make_tr(nchunks, chunk_rows):
        return functools.partial(
            pl.kernel,
            mesh=mesh,
            compiler_params=pltpu.CompilerParams(
                needs_layout_passes=False,
                use_tc_tiling_on_sc=False,
            ),
            out_type=jax.ShapeDtypeStruct(
                (nchunks * chunk_rows * IMG, LB, rows // LB), jnp.float32),
            scratch_types=[
                pltpu.VMEM((16, chunk_rows, IMG), jnp.float32),
                pltpu.VMEM((16, chunk_rows, IMG), jnp.float32),
                pltpu.VMEM((chunk_rows * IMG, LANES), jnp.float32),
                pltpu.VMEM((chunk_rows * IMG, LANES), jnp.float32),
                pltpu.SemaphoreType.DMA,
                pltpu.SemaphoreType.DMA,
                pltpu.SemaphoreType.DMA,
                pltpu.SemaphoreType.DMA,
            ],
        )(_make_tr_body(nchunks, chunk_rows))

    xta = make_tr(2, 16)(input_data[:, :, :32]).reshape(
        32 * IMG * LB, rows // LB)
    xtb = make_tr(3, 12)(input_data[:, :, 28:]).reshape(
        36 * IMG * LB, rows // LB)
    run = functools.partial(
        pl.kernel,
        mesh=mesh,
        compiler_params=pltpu.CompilerParams(
            needs_layout_passes=False,
            use_tc_tiling_on_sc=False,
        ),
        out_type=jax.ShapeDtypeStruct((OUT_ROWS * LB, rows // LB),
                                      jnp.float32),
        scratch_types=[
            pltpu.VMEM((NQ_PAD,), jnp.int32),
            pltpu.VMEM((NCHUNK, CROWS), jnp.int32),
            pltpu.VMEM((2 * NCHUNK, 128), jnp.int32),
            pltpu.VMEM((CGROWS, rows // LB), jnp.float32),
            pltpu.VMEM((CGROWS, rows // LB), jnp.float32),
            pltpu.VMEM((CGROWS, rows // LB), jnp.float32),
            pltpu.SemaphoreType.DMA,
            pltpu.SemaphoreType.DMA,
            pltpu.SemaphoreType.DMA,
            pltpu.SemaphoreType.DMA,
            pltpu.SemaphoreType.DMA,
            pltpu.SemaphoreType.DMA,
        ],
    )(_sc_body)
    out4 = run(xta, xtb, mask_p)
    out2d = (out4.reshape(OUT_ROWS // 8, LB, 8, rows // LB)
             .transpose(0, 2, 1, 3)
             .reshape(OUT_ROWS, rows))
    return out2d.T.reshape(rows, NQ, T)


# speedup vs baseline: 1.1999x; 1.1999x over previous
import functools

import jax
import jax.numpy as jnp
from jax import lax
from jax.experimental import pallas as pl
from jax.experimental.pallas import tpu as pltpu
from jax.experimental.pallas import tpu_sc as plsc

WINDOW = 61
PSIZE = 4
IMG = 64
NQ = 961
NQ_PAD = 976
T = PSIZE * PSIZE
OUT_ROWS = NQ * T
LANES = 16
NW = 32
CQ = 4
CROWS = CQ * T
NCHUNK = 8
LB = 4
CGROWS = CROWS * LB


def _tr_body(x4_hbm, xt_hbm, xv0, xv1, tb0, tb1,
             in_sem0, in_sem1, out_sem0, out_sem1):
    nc = 2
    wid = lax.axis_index("s") * nc + lax.axis_index("c")
    b = wid // 2
    c0 = (wid % 2) * 16
    tn = wid // 8
    ln0 = (wid % 8) * 16

    iota = lax.broadcasted_iota(jnp.int32, (LANES,), 0)
    kv = [jnp.broadcast_to(jnp.int32(k), (LANES,)) for k in range(16)]

    xvs = (xv0, xv1)
    tbs = (tb0, tb1)
    in_sems = (in_sem0, in_sem1)
    out_sems = (out_sem0, out_sem1)

    def start_in(ch, bb):
        cp = pltpu.make_async_copy(
            x4_hbm.at[b, pl.ds(c0, 16), pl.ds(ch * 16, 16)],
            xvs[bb], in_sems[bb])
        cp.start()
        return cp

    copies_in = [None, None]
    copies_out = [None, None]
    copies_in[0] = start_in(0, 0)
    for ch in range(4):
        bb = ch % 2
        nb = (ch + 1) % 2
        if ch + 1 < 4:
            if copies_out[nb] is not None:
                copies_out[nb].wait()
                copies_out[nb] = None
            copies_in[nb] = start_in(ch + 1, nb)
        copies_in[bb].wait()
        xv = xvs[bb]
        tb = tbs[bb]

        @plsc.parallel_loop(0, 1024 // LANES, unroll=4)
        def eg_step(eg):
            e0 = eg * LANES
            rr = lax.shift_right_logical(e0, 6)
            cc0 = e0 & 63
            ev = e0 + iota
            for k in range(16):
                vals = xv[k, rr, pl.ds(cc0, LANES)]
                plsc.store_scatter(tb, [ev, kv[k]], vals)

        copies_out[bb] = pltpu.make_async_copy(
            tb, xt_hbm.at[pl.ds(ch * 1024, 1024), tn, pl.ds(ln0, LANES)],
            out_sems[bb])
        copies_out[bb].start()
    for bb in range(2):
        if copies_out[bb] is not None:
            copies_out[bb].wait()


def _sc_body(xt_hbm, mask_hbm, out_hbm, mask_v, src_v, src2_v,
             buf0, buf1, buf2,
             in_sem0, in_sem1, in_sem2, out_sem0, out_sem1, out_sem2):
    nc = 2
    wid = lax.axis_index("s") * nc + lax.axis_index("c")
    q_start = (NQ * wid) // NW
    q_end = (NQ * (wid + 1)) // NW

    iota = lax.broadcasted_iota(jnp.int32, (LANES,), 0)
    off_v = lax.shift_right_logical(iota, 2) * IMG + (iota & 3)

    pltpu.sync_copy(mask_hbm, mask_v)

    def build_idx(c):
        qc = jnp.minimum(q_start + CQ * c, q_end - CQ)
        for gq in range(CQ):
            qv = jnp.broadcast_to(qc + gq, (LANES,))
            w = plsc.load_gather(mask_v, [qv])
            rr = lax.shift_right_logical(w * 68760, 22)
            base = w + 3 * rr
            src_v[c, pl.ds(gq * LANES, LANES)] = base + off_v

        def build(jg, _, c=c):
            jv = jg * LANES + iota
            tp = lax.shift_right_logical(jv, 5)
            tn = lax.shift_right_logical(jv, 3) & 3
            sp = jv & 7
            p_local = tp * 8 + sp
            cv = jnp.broadcast_to(jnp.int32(c), (LANES,))
            sr = plsc.load_gather(src_v, [cv, p_local])
            h = lax.shift_right_logical(jv, 7)
            jr = jv & 127
            plsc.store_scatter(src2_v, [2 * c + h, jr], sr * LB + tn)
            return 0

        lax.fori_loop(0, CGROWS // LANES, build, 0)

    bufs = (buf0, buf1, buf2)
    in_sems = (in_sem0, in_sem1, in_sem2)
    out_sems = (out_sem0, out_sem1, out_sem2)

    def band0(c):
        qc = jnp.minimum(q_start + CQ * c, q_end - CQ)
        return qc * (T // 8) * (LB * 8)

    def start_gather(c, b):
        copies = (
            pltpu.make_async_copy(
                xt_hbm.at[src2_v.at[2 * c]],
                bufs[b].at[pl.ds(0, 128)], in_sems[b]),
            pltpu.make_async_copy(
                xt_hbm.at[src2_v.at[2 * c + 1]],
                bufs[b].at[pl.ds(128, 128)], in_sems[b]),
        )
        for cp in copies:
            cp.start()
        return copies

    copies_in = [None, None, None]
    copies_out = [None, None, None]
    build_idx(0)
    copies_in[0] = start_gather(0, 0)
    build_idx(1)
    copies_in[1] = start_gather(1, 1)
    for c in range(NCHUNK):
        b = c % 3
        if c + 2 < NCHUNK:
            pb = (c + 2) % 3
            if copies_out[pb] is not None:
                copies_out[pb].wait()
                copies_out[pb] = None
            build_idx(c + 2)
            copies_in[pb] = start_gather(c + 2, pb)
        for cp in copies_in[b]:
            cp.wait()
        copies_out[b] = pltpu.make_async_copy(
            bufs[b], out_hbm.at[pl.ds(band0(c), CGROWS)], out_sems[b])
        copies_out[b].start()
    for b in range(3):
        if copies_out[b] is not None:
            copies_out[b].wait()


def kernel(input_data, mask):
    B, C, H, W = input_data.shape
    rows = B * C
    mask_p = jnp.pad(mask.astype(jnp.int32), (0, NQ_PAD - NQ))

    mesh = plsc.VectorSubcoreMesh(core_axis_name="c", subcore_axis_name="s")
    run_tr = functools.partial(
        pl.kernel,
        mesh=mesh,
        compiler_params=pltpu.CompilerParams(
            needs_layout_passes=False,
            use_tc_tiling_on_sc=False,
        ),
        out_type=jax.ShapeDtypeStruct((H * W, LB, rows // LB), jnp.float32),
        scratch_types=[
            pltpu.VMEM((16, 16, IMG), jnp.float32),
            pltpu.VMEM((16, 16, IMG), jnp.float32),
            pltpu.VMEM((1024, LANES), jnp.float32),
            pltpu.VMEM((1024, LANES), jnp.float32),
            pltpu.SemaphoreType.DMA,
            pltpu.SemaphoreType.DMA,
            pltpu.SemaphoreType.DMA,
            pltpu.SemaphoreType.DMA,
        ],
    )(_tr_body)
    xt4 = run_tr(input_data).reshape(H * W * LB, rows // LB)
    run = functools.partial(
        pl.kernel,
        mesh=mesh,
        compiler_params=pltpu.CompilerParams(
            needs_layout_passes=False,
            use_tc_tiling_on_sc=False,
        ),
        out_type=jax.ShapeDtypeStruct((OUT_ROWS * LB, rows // LB),
                                      jnp.float32),
        scratch_types=[
            pltpu.VMEM((NQ_PAD,), jnp.int32),
            pltpu.VMEM((NCHUNK, CROWS), jnp.int32),
            pltpu.VMEM((2 * NCHUNK, 128), jnp.int32),
            pltpu.VMEM((CGROWS, rows // LB), jnp.float32),
            pltpu.VMEM((CGROWS, rows // LB), jnp.float32),
            pltpu.VMEM((CGROWS, rows // LB), jnp.float32),
            pltpu.SemaphoreType.DMA,
            pltpu.SemaphoreType.DMA,
            pltpu.SemaphoreType.DMA,
            pltpu.SemaphoreType.DMA,
            pltpu.SemaphoreType.DMA,
            pltpu.SemaphoreType.DMA,
        ],
    )(_sc_body)
    out4 = run(xt4, mask_p)
    out2d = (out4.reshape(OUT_ROWS // 8, LB, 8, rows // LB)
             .transpose(0, 2, 1, 3)
             .reshape(OUT_ROWS, rows))
    return out2d.T.reshape(rows, NQ, T)
